# baseline (speedup 1.0000x reference)
import functools

import jax
import jax.numpy as jnp
from jax import lax
from jax.experimental import pallas as pl
from jax.experimental.pallas import tpu as pltpu
from jax.experimental.pallas import tpu_sc as plsc

E = 50000
T = 4
D = 128
H = 8
K = 16
DH = D // H


_NC = 2
_NS = 16
_NW = _NC * _NS
_CHUNK = 128
_NBUF = 2

_CHUNK_EDGES = (2000, 12000, 12000, 12000, 12000)
_C = len(_CHUNK_EDGES)
_CHUNK_BASE = tuple(sum(_CHUNK_EDGES[:i]) for i in range(_C))
_IDX_PAD = 64 * _CHUNK


def _make_sc_gather_body(base_chunk, nchunks, trips):
    def body(table_hbm, idx_hbm, out_hbm, idx_v, rows_v, sem_g, sem_s):
        w = lax.axis_index("s") * _NC + lax.axis_index("c")
        rem = nchunks - (trips - 1) * _NW
        start = base_chunk + w * (trips - 1) + jnp.minimum(w, rem)
        nch = (trips - 1) + jnp.where(w < rem, 1, 0)

        pltpu.sync_copy(idx_hbm.at[pl.ds(start * _CHUNK, trips * _CHUNK)],
                        idx_v)

        def step(j, carry):
            b = jax.lax.rem(j, _NBUF)

            @pl.when((j >= _NBUF) & (j - _NBUF < nch))
            def _():
                pltpu.make_async_copy(
                    out_hbm.at[pl.ds(0, _CHUNK)], rows_v.at[b], sem_s).wait()

            @pl.when(j < nch)
            def _():
                off = (w * (trips - 1) + jnp.minimum(w, rem) + j) * _CHUNK
                pltpu.async_copy(
                    table_hbm.at[idx_v.at[pl.ds(j * _CHUNK, _CHUNK)]],
                    rows_v.at[b], sem_g).wait()
                pltpu.async_copy(rows_v.at[b],
                                 out_hbm.at[pl.ds(off, _CHUNK)], sem_s)

            return carry

        lax.fori_loop(0, trips + _NBUF, step, 0)

    return body


def _sc_gather(c, table, idx_pad):
    n_edges = _CHUNK_EDGES[c]
    nchunks = n_edges * K // _CHUNK
    trips = -(-nchunks // _NW)
    base_chunk = _CHUNK_BASE[c] * K // _CHUNK
    mesh = plsc.VectorSubcoreMesh(core_axis_name="c", subcore_axis_name="s")
    f = pl.kernel(
        _make_sc_gather_body(base_chunk, nchunks, trips),
        out_type=jax.ShapeDtypeStruct((n_edges * K, D), jnp.float32),
        mesh=mesh,
        scratch_types=[
            pltpu.VMEM((trips * _CHUNK,), jnp.int32),
            pltpu.VMEM((_NBUF, _CHUNK, D), jnp.float32),
            pltpu.SemaphoreType.DMA,
            pltpu.SemaphoreType.DMA,
        ],
    )
    return f(table, idx_pad)


_B = 1000


def _tc_body(x_ref, xn_ref, wq_ref, wk_ref, wv_ref, wo_ref, bo_ref,
             ln1g_ref, ln1b_ref, ln2g_ref, ln2b_ref, w1_ref, b1_ref,
             w2_ref, b2_ref, bd_ref, out_ref):
    f32 = jnp.float32
    xb = x_ref[...]
    xn = xn_ref[...]
    wq = wq_ref[...]
    wk = wk_ref[...]
    wv = wv_ref[...]
    wo = wo_ref[...]
    bo = bo_ref[...]
    w1 = w1_ref[...]
    b1 = b1_ref[...]
    w2 = w2_ref[...]
    b2 = b2_ref[...]
    bd = bd_ref[...]
    ln1g = ln1g_ref[...]
    ln1b = ln1b_ref[...]
    ln2g = ln2g_ref[...]
    ln2b = ln2b_ref[...]

    kn3 = jnp.dot(xn, wk, preferred_element_type=f32).reshape(_B, K, D)
    vn = jnp.dot(xn, wv, preferred_element_type=f32)

    def layer_norm(v, g, b):
        mu = jnp.mean(v, axis=-1, keepdims=True)
        var = jnp.mean((v - mu) ** 2, axis=-1, keepdims=True)
        return (v - mu) * lax.rsqrt(var + 1e-5) * g + b

    for t in range(T):
        xt = xb[:, t * D:(t + 1) * D]
        qt = jnp.dot(xt, wq, preferred_element_type=f32)
        p3 = qt[:, None, :] * kn3
        sbc = jnp.dot(p3.reshape(_B * K, D), bd,
                      preferred_element_type=f32)
        wts = jnp.exp(jnp.clip(sbc, -60.0, 60.0))
        t4 = (wts * vn).reshape(_B, K, D)
        t4 = t4[:, : K // 2, :] + t4[:, K // 2:, :]
        znum = jnp.sum(t4, axis=1)
        w4 = wts.reshape(_B, K, D)
        w4 = w4[:, : K // 2, :] + w4[:, K // 2:, :]
        den = jnp.sum(w4, axis=1)
        z = znum / den
        ot = jnp.dot(z, wo, preferred_element_type=f32) + bo
        h = layer_norm(xt + ot, ln1g, ln1b)
        ff = jnp.dot(jax.nn.relu(jnp.dot(h, w1, preferred_element_type=f32)
                                 + b1), w2, preferred_element_type=f32) + b2
        h2 = layer_norm(h + ff, ln2g, ln2b)
        out_ref[:, t * D:(t + 1) * D] = h2


def _tc_fused(c, x2, xnbr, Wq, Wk, Wv, Wo, bo, ln1_g, ln1_b, ln2_g,
              ln2_b, W1, b1, W2, b2, bd):
    n_edges = _CHUNK_EDGES[c]
    base_blk = _CHUNK_BASE[c] // _B
    rep = lambda shape: pl.BlockSpec(shape, lambda i: (0,) * len(shape))
    grid_spec = pl.GridSpec(
        grid=(n_edges // _B,),
        in_specs=[
            pl.BlockSpec((_B, T * D), lambda i: (i + base_blk, 0)),
            pl.BlockSpec((_B * K, D), lambda i: (i, 0)),
            rep((D, D)), rep((D, D)), rep((D, D)), rep((D, D)), rep((D,)),
            rep((D,)), rep((D,)), rep((D,)), rep((D,)),
            rep((D, 4 * D)), rep((4 * D,)), rep((4 * D, D)), rep((D,)),
            rep((D, D)),
        ],
        out_specs=pl.BlockSpec((_B, T * D), lambda i: (i, 0)),
    )
    return pl.pallas_call(
        _tc_body,
        grid_spec=grid_spec,
        out_shape=jax.ShapeDtypeStruct((n_edges, T * D), jnp.float32),
        compiler_params=pltpu.CompilerParams(
            dimension_semantics=("arbitrary",),
        ),
    )(x2, xnbr, Wq, Wk, Wv, Wo, bo, ln1_g, ln1_b, ln2_g, ln2_b,
      W1, b1, W2, b2, bd)


def kernel(x, neighbor_index, Wq, Wk, Wv, Wo, bo, ln1_g, ln1_b, ln2_g,
           ln2_b, W1, b1, W2, b2):
    x2 = x.reshape(E, T * D)
    table = x.reshape(E * T, D)
    flat_idx = (neighbor_index * T).reshape(E * K)
    idx_pad = jnp.concatenate(
        [flat_idx, jnp.zeros((_IDX_PAD,), jnp.int32)])

    dd = jnp.arange(D, dtype=jnp.int32)
    bd = jnp.where(dd[:, None] // DH == dd[None, :] // DH,
                   1.0 / (DH ** 0.5), 0.0).astype(jnp.float32)
    outs = []
    for c in range(_C):
        xnbr_c = _sc_gather(c, table, idx_pad)
        outs.append(_tc_fused(c, x2, xnbr_c, Wq, Wk, Wv, Wo, bo, ln1_g,
                              ln1_b, ln2_g, ln2_b, W1, b1, W2, b2, bd))
    return jnp.concatenate(outs, axis=0).reshape(E, T, D)


# speedup vs baseline: 1.0277x; 1.0277x over previous
import functools

import jax
import jax.numpy as jnp
from jax import lax
from jax.experimental import pallas as pl
from jax.experimental.pallas import tpu as pltpu
from jax.experimental.pallas import tpu_sc as plsc

E = 50000
T = 4
D = 128
H = 8
K = 16
DH = D // H


_NC = 2
_NS = 16
_NW = _NC * _NS
_CHUNK = 128
_NBUF = 2

_CHUNK_EDGES = (2000, 6000, 14000, 14000, 14000)
_C = len(_CHUNK_EDGES)
_CHUNK_BASE = tuple(sum(_CHUNK_EDGES[:i]) for i in range(_C))
_IDX_PAD = 64 * _CHUNK


def _make_sc_gather_body(base_chunk, nchunks, trips):
    def body(table_hbm, idx_hbm, out_hbm, idx_v, rows_v, sem_g, sem_s):
        w = lax.axis_index("s") * _NC + lax.axis_index("c")
        rem = nchunks - (trips - 1) * _NW
        start = base_chunk + w * (trips - 1) + jnp.minimum(w, rem)
        nch = (trips - 1) + jnp.where(w < rem, 1, 0)

        pltpu.sync_copy(idx_hbm.at[pl.ds(start * _CHUNK, trips * _CHUNK)],
                        idx_v)

        def step(j, carry):
            b = jax.lax.rem(j, _NBUF)

            @pl.when((j >= _NBUF) & (j - _NBUF < nch))
            def _():
                pltpu.make_async_copy(
                    out_hbm.at[pl.ds(0, _CHUNK)], rows_v.at[b], sem_s).wait()

            @pl.when(j < nch)
            def _():
                off = (w * (trips - 1) + jnp.minimum(w, rem) + j) * _CHUNK
                pltpu.async_copy(
                    table_hbm.at[idx_v.at[pl.ds(j * _CHUNK, _CHUNK)]],
                    rows_v.at[b], sem_g).wait()
                pltpu.async_copy(rows_v.at[b],
                                 out_hbm.at[pl.ds(off, _CHUNK)], sem_s)

            return carry

        lax.fori_loop(0, trips + _NBUF, step, 0)

    return body


def _sc_gather(c, table, idx_pad):
    n_edges = _CHUNK_EDGES[c]
    nchunks = n_edges * K // _CHUNK
    trips = -(-nchunks // _NW)
    base_chunk = _CHUNK_BASE[c] * K // _CHUNK
    mesh = plsc.VectorSubcoreMesh(core_axis_name="c", subcore_axis_name="s")
    f = pl.kernel(
        _make_sc_gather_body(base_chunk, nchunks, trips),
        out_type=jax.ShapeDtypeStruct((n_edges * K, D), jnp.float32),
        mesh=mesh,
        scratch_types=[
            pltpu.VMEM((trips * _CHUNK,), jnp.int32),
            pltpu.VMEM((_NBUF, _CHUNK, D), jnp.float32),
            pltpu.SemaphoreType.DMA,
            pltpu.SemaphoreType.DMA,
        ],
    )
    return f(table, idx_pad)


_B = 1000


def _tc_body(x_ref, xn_ref, wq_ref, wk_ref, wv_ref, wo_ref, bo_ref,
             ln1g_ref, ln1b_ref, ln2g_ref, ln2b_ref, w1_ref, b1_ref,
             w2_ref, b2_ref, bd_ref, out_ref):
    f32 = jnp.float32
    xb = x_ref[...]
    xn = xn_ref[...]
    wq = wq_ref[...]
    wk = wk_ref[...]
    wv = wv_ref[...]
    wo = wo_ref[...]
    bo = bo_ref[...]
    w1 = w1_ref[...]
    b1 = b1_ref[...]
    w2 = w2_ref[...]
    b2 = b2_ref[...]
    bd = bd_ref[...]
    ln1g = ln1g_ref[...]
    ln1b = ln1b_ref[...]
    ln2g = ln2g_ref[...]
    ln2b = ln2b_ref[...]

    kn3 = jnp.dot(xn, wk, preferred_element_type=f32).reshape(_B, K, D)
    vn = jnp.dot(xn, wv, preferred_element_type=f32)

    def layer_norm(v, g, b):
        mu = jnp.mean(v, axis=-1, keepdims=True)
        var = jnp.mean((v - mu) ** 2, axis=-1, keepdims=True)
        return (v - mu) * lax.rsqrt(var + 1e-5) * g + b

    for t in range(T):
        xt = xb[:, t * D:(t + 1) * D]
        qt = jnp.dot(xt, wq, preferred_element_type=f32)
        p3 = qt[:, None, :] * kn3
        sbc = jnp.dot(p3.reshape(_B * K, D), bd,
                      preferred_element_type=f32)
        wts = jnp.exp(jnp.clip(sbc, -60.0, 60.0))
        t4 = (wts * vn).reshape(_B, K, D)
        t4 = t4[:, : K // 2, :] + t4[:, K // 2:, :]
        znum = jnp.sum(t4, axis=1)
        w4 = wts.reshape(_B, K, D)
        w4 = w4[:, : K // 2, :] + w4[:, K // 2:, :]
        den = jnp.sum(w4, axis=1)
        z = znum / den
        ot = jnp.dot(z, wo, preferred_element_type=f32) + bo
        h = layer_norm(xt + ot, ln1g, ln1b)
        ff = jnp.dot(jax.nn.relu(jnp.dot(h, w1, preferred_element_type=f32)
                                 + b1), w2, preferred_element_type=f32) + b2
        h2 = layer_norm(h + ff, ln2g, ln2b)
        out_ref[:, t * D:(t + 1) * D] = h2


def _tc_fused(c, x2, xnbr, Wq, Wk, Wv, Wo, bo, ln1_g, ln1_b, ln2_g,
              ln2_b, W1, b1, W2, b2, bd):
    n_edges = _CHUNK_EDGES[c]
    base_blk = _CHUNK_BASE[c] // _B
    rep = lambda shape: pl.BlockSpec(shape, lambda i: (0,) * len(shape))
    grid_spec = pl.GridSpec(
        grid=(n_edges // _B,),
        in_specs=[
            pl.BlockSpec((_B, T * D), lambda i: (i + base_blk, 0)),
            pl.BlockSpec((_B * K, D), lambda i: (i, 0)),
            rep((D, D)), rep((D, D)), rep((D, D)), rep((D, D)), rep((D,)),
            rep((D,)), rep((D,)), rep((D,)), rep((D,)),
            rep((D, 4 * D)), rep((4 * D,)), rep((4 * D, D)), rep((D,)),
            rep((D, D)),
        ],
        out_specs=pl.BlockSpec((_B, T * D), lambda i: (i, 0)),
    )
    return pl.pallas_call(
        _tc_body,
        grid_spec=grid_spec,
        out_shape=jax.ShapeDtypeStruct((n_edges, T * D), jnp.float32),
        compiler_params=pltpu.CompilerParams(
            dimension_semantics=("arbitrary",),
        ),
    )(x2, xnbr, Wq, Wk, Wv, Wo, bo, ln1_g, ln1_b, ln2_g, ln2_b,
      W1, b1, W2, b2, bd)


def kernel(x, neighbor_index, Wq, Wk, Wv, Wo, bo, ln1_g, ln1_b, ln2_g,
           ln2_b, W1, b1, W2, b2):
    x2 = x.reshape(E, T * D)
    table = x.reshape(E * T, D)
    flat_idx = (neighbor_index * T).reshape(E * K)
    idx_pad = jnp.concatenate(
        [flat_idx, jnp.zeros((_IDX_PAD,), jnp.int32)])

    dd = jnp.arange(D, dtype=jnp.int32)
    bd = jnp.where(dd[:, None] // DH == dd[None, :] // DH,
                   1.0 / (DH ** 0.5), 0.0).astype(jnp.float32)
    outs = []
    for c in range(_C):
        xnbr_c = _sc_gather(c, table, idx_pad)
        outs.append(_tc_fused(c, x2, xnbr_c, Wq, Wk, Wv, Wo, bo, ln1_g,
                              ln1_b, ln2_g, ln2_b, W1, b1, W2, b2, bd))
    return jnp.concatenate(outs, axis=0).reshape(E, T, D)
